# Initial kernel scaffold; baseline (speedup 1.0000x reference)
import jax
import jax.numpy as jnp
from jax.experimental import pallas as pl


def kernel(x, edge_index, W1, b1, W2, attn_l, attn_r, gat_bias):
    raise NotImplementedError("write your pallas kernel here")


# speedup vs baseline: 5.7679x; 5.7679x over previous
import functools

import jax
import jax.numpy as jnp
from jax import lax
from jax.experimental import pallas as pl
from jax.experimental.pallas import tpu as pltpu
from jax.experimental.pallas import tpu_sc as plsc

N = 10000
E = 160000
D_IN = 256
D_HID = 512
H = 8
F = 256
FH = 128
NEG = 0.2

NC = 2
NS = 16
L = 16

BN = 400
GRID_N = N // BN

EPW = E // (NC * NS)
CA = 40
NCA = EPW // CA

EPS = E // NS
CB = 16
NCB = EPS // CB

TPR = 640
ZCA = 40
RW = 16

_f32 = jnp.float32
_i32 = jnp.int32


def _dense_body(x_ref, w1_ref, b1_ref, w2_ref, al_ref, ar_ref,
                fa_ref, fb_ref, elp_ref, erp_ref, mx_ref):
    i = pl.program_id(0)
    h = jnp.maximum(
        jnp.dot(x_ref[...], w1_ref[...], preferred_element_type=_f32)
        + b1_ref[...], 0.0)
    feat = jnp.dot(h, w2_ref[...], preferred_element_type=_f32)
    ft = feat.reshape(BN, H, F)
    el = jnp.sum(ft * al_ref[...][None], axis=-1)
    er = jnp.sum(ft * ar_ref[...][None], axis=-1)
    z = jnp.zeros((BN, FH - H), _f32)
    elp_ref[...] = jnp.concatenate([el, z], axis=1)
    erp_ref[...] = jnp.concatenate([er, z], axis=1)
    fa_ref[...] = ft[:, :, :FH].reshape(BN, H * FH)
    fb_ref[...] = ft[:, :, FH:].reshape(BN, H * FH)
    bm = jnp.stack([jnp.full((FH,), jnp.max(el)), jnp.full((FH,), jnp.max(er))])
    prev = jnp.where(i == 0, jnp.full((2, FH), -jnp.inf, _f32), mx_ref[...])
    mx_ref[...] = jnp.maximum(prev, bm)


def _dense(x, W1, b1, W2, attn_l, attn_r):
    return pl.pallas_call(
        _dense_body,
        grid=(GRID_N,),
        in_specs=[
            pl.BlockSpec((BN, D_IN), lambda i: (i, 0)),
            pl.BlockSpec((D_IN, D_HID), lambda i: (0, 0)),
            pl.BlockSpec((1, D_HID), lambda i: (0, 0)),
            pl.BlockSpec((D_HID, H * F), lambda i: (0, 0)),
            pl.BlockSpec((H, F), lambda i: (0, 0)),
            pl.BlockSpec((H, F), lambda i: (0, 0)),
        ],
        out_specs=[
            pl.BlockSpec((BN, H * FH), lambda i: (i, 0)),
            pl.BlockSpec((BN, H * FH), lambda i: (i, 0)),
            pl.BlockSpec((BN, FH), lambda i: (i, 0)),
            pl.BlockSpec((BN, FH), lambda i: (i, 0)),
            pl.BlockSpec((2, FH), lambda i: (0, 0)),
        ],
        out_shape=[
            jax.ShapeDtypeStruct((N, H * FH), _f32),
            jax.ShapeDtypeStruct((N, H * FH), _f32),
            jax.ShapeDtypeStruct((N, FH), _f32),
            jax.ShapeDtypeStruct((N, FH), _f32),
            jax.ShapeDtypeStruct((2, FH), _f32),
        ],
    )(x, W1, b1.reshape(1, D_HID), W2, attn_l, attn_r)


def _rden_body(p0_ref, p1_ref, o_ref):
    o_ref[...] = (1.0 / H) / (p0_ref[...] + p1_ref[...] + 1e-30)


def _rden(p0, p1):
    return pl.pallas_call(
        _rden_body,
        grid=(5,),
        in_specs=[pl.BlockSpec((N // 5, FH), lambda i: (i, 0)),
                  pl.BlockSpec((N // 5, FH), lambda i: (i, 0))],
        out_specs=pl.BlockSpec((N // 5, FH), lambda i: (i, 0)),
        out_shape=jax.ShapeDtypeStruct((N, FH), _f32),
    )(p0, p1)


@functools.lru_cache(maxsize=None)
def _sc_kernels():
  mesh = plsc.VectorSubcoreMesh(core_axis_name="c", subcore_axis_name="s",
                                num_cores=NC, num_subcores=NS)

  @functools.partial(
      pl.kernel,
      out_type=[
          jax.ShapeDtypeStruct((E, L), _f32),
          jax.ShapeDtypeStruct((NC * N, FH), _f32),
      ],
      mesh=mesh,
      scratch_types=[
          pltpu.VMEM((CA,), _i32),
          pltpu.VMEM((CA,), _i32),
          pltpu.VMEM((CA, FH), _f32),
          pltpu.VMEM((CA, FH), _f32),
          pltpu.VMEM((CA, L), _f32),
          pltpu.VMEM((CA, FH), _f32),
          pltpu.VMEM((2, FH), _f32),
          pltpu.VMEM((L,), _f32),
          pltpu.VMEM_SHARED((N, FH), _f32),
          pltpu.SemaphoreType.DMA,
      ],
  )
  def phase_a(elp, erp, mx, srch, dsth, eeh, pdenh,
              sidx, didx, elr, err_, eebuf, eewide, mxbuf, bref, pdacc, sem):
    c = lax.axis_index("c")
    s = lax.axis_index("s")
    w = s * NC + c

    pltpu.sync_copy(mx, mxbuf)
    bref[...] = mxbuf[0, pl.ds(0, L)] + mxbuf[1, pl.ds(0, L)]

    def _zrow(j, _):
        for k in range(FH // L):
            eewide[j, pl.ds(k * L, L)] = jnp.zeros((L,), _f32)
        return 0
    lax.fori_loop(0, CA, _zrow, 0)

    for r in range(TPR // ZCA):
        off = s * TPR + r * ZCA

        @pl.when(off < N)
        def _(off=off):
            pltpu.sync_copy(eewide.at[pl.ds(0, ZCA)],
                            pdacc.at[pl.ds(off, ZCA)])
    plsc.subcore_barrier()

    def _chunk(ci, _):
        base = w * EPW + ci * CA
        pltpu.sync_copy(srch.at[pl.ds(base, CA)], sidx)
        pltpu.sync_copy(dsth.at[pl.ds(base, CA)], didx)
        pltpu.async_copy(elp.at[sidx], elr, sem).wait()
        pltpu.async_copy(erp.at[didx], err_, sem).wait()
        b = bref[...]

        def _edge(j, _):
            ev = elr[j, pl.ds(0, L)] + err_[j, pl.ds(0, L)]
            ee = jnp.exp(jnp.maximum(ev, NEG * ev) - b)
            eebuf[j, :] = ee
            eewide[j, pl.ds(0, L)] = ee
            return 0
        lax.fori_loop(0, CA, _edge, 0)
        pltpu.sync_copy(eebuf, eeh.at[pl.ds(base, CA)])
        pltpu.sync_copy(eewide, pdacc.at[didx], add=True)
        return 0
    lax.fori_loop(0, NCA, _chunk, 0)

    plsc.subcore_barrier()
    for r in range(TPR // ZCA):
        off = s * TPR + r * ZCA

        @pl.when(off < N)
        def _(off=off):
            pltpu.sync_copy(pdacc.at[pl.ds(off, ZCA)],
                            pdenh.at[pl.ds(c * N + off, ZCA)])

  @functools.partial(
      pl.kernel,
      out_type=jax.ShapeDtypeStruct((NC * N, FH), _f32),
      mesh=mesh,
      scratch_types=[
          pltpu.VMEM((CB,), _i32),
          pltpu.VMEM((CB,), _i32),
          pltpu.VMEM((CB, L), _f32),
          pltpu.VMEM((CB, FH), _f32),
          pltpu.VMEM((CB, H * FH), _f32),
          pltpu.VMEM((CB, FH), _f32),
          pltpu.VMEM((H, FH), _f32),
          pltpu.VMEM((FH,), _f32),
          pltpu.VMEM_SHARED((N, FH), _f32),
          pltpu.SemaphoreType.DMA,
      ],
  )
  def phase_b(fah, fbh, eeh, rdenh, srch, dsth, gbh, outh,
              sidx, didx, eebuf, denb, featbuf, msgbuf,
              gbbuf, biasbuf, oacc, sem):
    c = lax.axis_index("c")
    s = lax.axis_index("s")

    def _zrow(j, _):
        for k in range(H):
            msgbuf[j, pl.ds(k * L, L)] = jnp.zeros((L,), _f32)
        return 0
    lax.fori_loop(0, RW, _zrow, 0)
    for r in range(TPR // RW):
        off = s * TPR + r * RW

        @pl.when(off < N)
        def _(off=off):
            pltpu.sync_copy(msgbuf, oacc.at[pl.ds(off, RW)])
    plsc.subcore_barrier()

    def _chunk(ci, _):
        base = s * EPS + ci * CB
        pltpu.sync_copy(srch.at[pl.ds(base, CB)], sidx)
        pltpu.sync_copy(dsth.at[pl.ds(base, CB)], didx)
        pltpu.sync_copy(eeh.at[pl.ds(base, CB)], eebuf)
        pltpu.async_copy(rdenh.at[didx], denb, sem).wait()

        @pl.when(c == 0)
        def _():
            pltpu.async_copy(fah.at[sidx], featbuf, sem).wait()

        @pl.when(c == 1)
        def _():
            pltpu.async_copy(fbh.at[sidx], featbuf, sem).wait()

        def _edge(j, _):
            wv = eebuf[j] * denb[j, pl.ds(0, L)]
            for k in range(H):
                m = jnp.zeros((L,), _f32)
                for h in range(H):
                    sv = jnp.full((L,), wv[h], _f32)
                    m = m + sv * featbuf[j, pl.ds(h * FH + k * L, L)]
                msgbuf[j, pl.ds(k * L, L)] = m
            return 0
        lax.fori_loop(0, CB, _edge, 0)
        pltpu.sync_copy(msgbuf, oacc.at[didx], add=True)
        return 0
    lax.fori_loop(0, NCB, _chunk, 0)

    plsc.subcore_barrier()

    pltpu.sync_copy(gbh.at[:, pl.ds(c * FH, FH)], gbbuf)
    for k in range(H):
        bv = jnp.zeros((L,), _f32)
        for h in range(H):
            bv = bv + gbbuf[h, pl.ds(k * L, L)]
        biasbuf[pl.ds(k * L, L)] = bv * (1.0 / H)

    for r in range(TPR // RW):
        off = s * TPR + r * RW

        @pl.when(off < N)
        def _(off=off):
            pltpu.sync_copy(oacc.at[pl.ds(off, RW)], msgbuf)

            def _brow(j, _):
                for k in range(H):
                    msgbuf[j, pl.ds(k * L, L)] = (msgbuf[j, pl.ds(k * L, L)]
                                                  + biasbuf[pl.ds(k * L, L)])
                return 0
            lax.fori_loop(0, RW, _brow, 0)
            pltpu.sync_copy(msgbuf, outh.at[pl.ds(c * N + off, RW)])

  return phase_a, phase_b


def kernel(x, edge_index, W1, b1, W2, attn_l, attn_r, gat_bias):
    phase_a, phase_b = _sc_kernels()
    fa, fb, elp, erp, mx = _dense(x, W1, b1, W2, attn_l, attn_r)
    src = edge_index[0]
    dst = edge_index[1]
    ee, pden = phase_a(elp, erp, mx, src, dst)
    rden = _rden(pden[:N], pden[N:])
    o2 = phase_b(fa, fb, ee, rden, src, dst, gat_bias)
    return jnp.concatenate([o2[:N], o2[N:]], axis=1)


# speedup vs baseline: 10.7697x; 1.8672x over previous
import functools

import jax
import jax.numpy as jnp
from jax import lax
from jax.experimental import pallas as pl
from jax.experimental.pallas import tpu as pltpu
from jax.experimental.pallas import tpu_sc as plsc

N = 10000
E = 160000
D_IN = 256
D_HID = 512
H = 8
F = 256
FH = 128
NEG = 0.2

NC = 2
NS = 16
L = 16

BN = 400
GRID_N = N // BN

EPW = E // (NC * NS)
CA = 40
NCA = EPW // CA

EPS = E // NS
CB = 16
NCB = EPS // CB

TPR = 640
ZCA = 40
RW = 16

_f32 = jnp.float32
_i32 = jnp.int32


def _dense_body(x_ref, w1_ref, b1_ref, w2_ref, al_ref, ar_ref,
                fa_ref, fb_ref, elp_ref, erp_ref, mx_ref):
    i = pl.program_id(0)
    h = jnp.maximum(
        jnp.dot(x_ref[...], w1_ref[...], preferred_element_type=_f32)
        + b1_ref[...], 0.0)
    feat = jnp.dot(h, w2_ref[...], preferred_element_type=_f32)
    ft = feat.reshape(BN, H, F)
    el = jnp.sum(ft * al_ref[...][None], axis=-1)
    er = jnp.sum(ft * ar_ref[...][None], axis=-1)
    z = jnp.zeros((BN, FH - H), _f32)
    elp_ref[...] = jnp.concatenate([el, z], axis=1)
    erp_ref[...] = jnp.concatenate([er, z], axis=1)
    fa_ref[...] = ft[:, :, :FH].reshape(BN, H * FH)
    fb_ref[...] = ft[:, :, FH:].reshape(BN, H * FH)
    bm = jnp.stack([jnp.full((FH,), jnp.max(el)), jnp.full((FH,), jnp.max(er))])
    prev = jnp.where(i == 0, jnp.full((2, FH), -jnp.inf, _f32), mx_ref[...])
    mx_ref[...] = jnp.maximum(prev, bm)


def _dense(x, W1, b1, W2, attn_l, attn_r):
    return pl.pallas_call(
        _dense_body,
        grid=(GRID_N,),
        in_specs=[
            pl.BlockSpec((BN, D_IN), lambda i: (i, 0)),
            pl.BlockSpec((D_IN, D_HID), lambda i: (0, 0)),
            pl.BlockSpec((1, D_HID), lambda i: (0, 0)),
            pl.BlockSpec((D_HID, H * F), lambda i: (0, 0)),
            pl.BlockSpec((H, F), lambda i: (0, 0)),
            pl.BlockSpec((H, F), lambda i: (0, 0)),
        ],
        out_specs=[
            pl.BlockSpec((BN, H * FH), lambda i: (i, 0)),
            pl.BlockSpec((BN, H * FH), lambda i: (i, 0)),
            pl.BlockSpec((BN, FH), lambda i: (i, 0)),
            pl.BlockSpec((BN, FH), lambda i: (i, 0)),
            pl.BlockSpec((2, FH), lambda i: (0, 0)),
        ],
        out_shape=[
            jax.ShapeDtypeStruct((N, H * FH), _f32),
            jax.ShapeDtypeStruct((N, H * FH), _f32),
            jax.ShapeDtypeStruct((N, FH), _f32),
            jax.ShapeDtypeStruct((N, FH), _f32),
            jax.ShapeDtypeStruct((2, FH), _f32),
        ],
    )(x, W1, b1.reshape(1, D_HID), W2, attn_l, attn_r)


def _rden_body(p0_ref, p1_ref, o_ref):
    o_ref[...] = (1.0 / H) / (p0_ref[...] + p1_ref[...] + 1e-30)


def _rden(p0, p1):
    return pl.pallas_call(
        _rden_body,
        grid=(5,),
        in_specs=[pl.BlockSpec((N // 5, FH), lambda i: (i, 0)),
                  pl.BlockSpec((N // 5, FH), lambda i: (i, 0))],
        out_specs=pl.BlockSpec((N // 5, FH), lambda i: (i, 0)),
        out_shape=jax.ShapeDtypeStruct((N, FH), _f32),
    )(p0, p1)


@functools.lru_cache(maxsize=None)
def _sc_kernels():
  mesh = plsc.VectorSubcoreMesh(core_axis_name="c", subcore_axis_name="s",
                                num_cores=NC, num_subcores=NS)

  @functools.partial(
      pl.kernel,
      out_type=[
          jax.ShapeDtypeStruct((E, L), _f32),
          jax.ShapeDtypeStruct((NC * N, FH), _f32),
      ],
      mesh=mesh,
      scratch_types=[
          pltpu.VMEM((CA,), _i32),
          pltpu.VMEM((CA,), _i32),
          pltpu.VMEM((CA, FH), _f32),
          pltpu.VMEM((CA, FH), _f32),
          pltpu.VMEM((CA, L), _f32),
          pltpu.VMEM((CA, FH), _f32),
          pltpu.VMEM((2, FH), _f32),
          pltpu.VMEM((L,), _f32),
          pltpu.VMEM_SHARED((N, FH), _f32),
          pltpu.SemaphoreType.DMA,
      ],
  )
  def phase_a(elp, erp, mx, srch, dsth, eeh, pdenh,
              sidx, didx, elr, err_, eebuf, eewide, mxbuf, bref, pdacc, sem):
    c = lax.axis_index("c")
    s = lax.axis_index("s")
    w = s * NC + c

    pltpu.sync_copy(mx, mxbuf)
    bref[...] = mxbuf[0, pl.ds(0, L)] + mxbuf[1, pl.ds(0, L)]

    def _zrow(j, _):
        for k in range(FH // L):
            eewide[j, pl.ds(k * L, L)] = jnp.zeros((L,), _f32)
        return 0
    lax.fori_loop(0, CA, _zrow, 0)

    for r in range(TPR // ZCA):
        off = s * TPR + r * ZCA

        @pl.when(off < N)
        def _(off=off):
            pltpu.sync_copy(eewide.at[pl.ds(0, ZCA)],
                            pdacc.at[pl.ds(off, ZCA)])
    plsc.subcore_barrier()

    def _chunk(ci, _):
        base = w * EPW + ci * CA
        pltpu.sync_copy(srch.at[pl.ds(base, CA)], sidx)
        pltpu.sync_copy(dsth.at[pl.ds(base, CA)], didx)
        pltpu.async_copy(elp.at[sidx], elr, sem).wait()
        pltpu.async_copy(erp.at[didx], err_, sem).wait()
        b = bref[...]

        def _edge(j, _):
            ev = elr[j, pl.ds(0, L)] + err_[j, pl.ds(0, L)]
            ee = jnp.exp(jnp.maximum(ev, NEG * ev) - b)
            eebuf[j, :] = ee
            eewide[j, pl.ds(0, L)] = ee
            return 0
        lax.fori_loop(0, CA, _edge, 0)
        pltpu.sync_copy(eebuf, eeh.at[pl.ds(base, CA)])
        pltpu.sync_copy(eewide, pdacc.at[didx], add=True)
        return 0
    lax.fori_loop(0, NCA, _chunk, 0)

    plsc.subcore_barrier()
    for r in range(TPR // ZCA):
        off = s * TPR + r * ZCA

        @pl.when(off < N)
        def _(off=off):
            pltpu.sync_copy(pdacc.at[pl.ds(off, ZCA)],
                            pdenh.at[pl.ds(c * N + off, ZCA)])

  @functools.partial(
      pl.kernel,
      out_type=jax.ShapeDtypeStruct((NC * N, FH), _f32),
      mesh=mesh,
      scratch_types=[
          pltpu.VMEM((CB,), _i32),
          pltpu.VMEM((CB,), _i32),
          pltpu.VMEM((CB,), _i32),
          pltpu.VMEM((CB,), _i32),
          pltpu.VMEM((CB, L), _f32),
          pltpu.VMEM((CB, L), _f32),
          pltpu.VMEM((CB, FH), _f32),
          pltpu.VMEM((CB, FH), _f32),
          pltpu.VMEM((CB, H * FH), _f32),
          pltpu.VMEM((CB, H * FH), _f32),
          pltpu.VMEM((CB, FH), _f32),
          pltpu.VMEM((CB, FH), _f32),
          pltpu.VMEM((H, FH), _f32),
          pltpu.VMEM((FH,), _f32),
          pltpu.VMEM_SHARED((N, FH), _f32),
          pltpu.SemaphoreType.DMA,
          pltpu.SemaphoreType.DMA,
          pltpu.SemaphoreType.DMA,
          pltpu.SemaphoreType.DMA,
      ],
  )
  def phase_b(fah, fbh, eeh, rdenh, srch, dsth, gbh, outh,
              sidx0, sidx1, didx0, didx1, eeb0, eeb1, den0, den1,
              fb0, fb1, mb0, mb1, gbbuf, biasbuf, oacc,
              isem0, isem1, gsem0, gsem1):
    c = lax.axis_index("c")
    s = lax.axis_index("s")
    sidx = [sidx0, sidx1]
    didx = [didx0, didx1]
    eeb = [eeb0, eeb1]
    den = [den0, den1]
    fbuf = [fb0, fb1]
    mbuf = [mb0, mb1]
    isem = [isem0, isem1]
    gsem = [gsem0, gsem1]
    msgbuf = mb0

    def _zrow(j, _):
        for k in range(H):
            msgbuf[j, pl.ds(k * L, L)] = jnp.zeros((L,), _f32)
        return 0
    lax.fori_loop(0, RW, _zrow, 0)
    for r in range(TPR // RW):
        off = s * TPR + r * RW

        @pl.when(off < N)
        def _(off=off):
            pltpu.sync_copy(msgbuf, oacc.at[pl.ds(off, RW)])
    plsc.subcore_barrier()

    def _base(ci):
        return s * EPS + ci * CB

    def _fire_idx(ci, b):
        pltpu.async_copy(srch.at[pl.ds(_base(ci), CB)], sidx[b], isem[b])
        pltpu.async_copy(dsth.at[pl.ds(_base(ci), CB)], didx[b], isem[b])

    def _wait_idx(b):
        pltpu.make_async_copy(srch.at[pl.ds(0, CB)], sidx[b], isem[b]).wait()
        pltpu.make_async_copy(dsth.at[pl.ds(0, CB)], didx[b], isem[b]).wait()

    def _fire_gathers(ci, b):
        pltpu.async_copy(eeh.at[pl.ds(_base(ci), CB)], eeb[b], gsem[b])
        pltpu.async_copy(rdenh.at[didx[b]], den[b], gsem[b])

        @pl.when(c == 0)
        def _():
            pltpu.async_copy(fah.at[sidx[b]], fbuf[b], gsem[b])

        @pl.when(c == 1)
        def _():
            pltpu.async_copy(fbh.at[sidx[b]], fbuf[b], gsem[b])

    def _wait_gathers(b):
        pltpu.make_async_copy(eeh.at[pl.ds(0, CB)], eeb[b], gsem[b]).wait()
        pltpu.make_async_copy(rdenh.at[didx[b]], den[b], gsem[b]).wait()
        pltpu.make_async_copy(fah.at[sidx[b]], fbuf[b], gsem[b]).wait()

    def _compute(b):
        featbuf = fbuf[b]
        mb = mbuf[b]
        eebuf = eeb[b]
        denb = den[b]

        def _edge(j, _):
            wv = eebuf[j] * denb[j, pl.ds(0, L)]
            for k in range(H):
                m = jnp.zeros((L,), _f32)
                for h in range(H):
                    sv = jnp.full((L,), wv[h], _f32)
                    m = m + sv * featbuf[j, pl.ds(h * FH + k * L, L)]
                mb[j, pl.ds(k * L, L)] = m
            return 0
        lax.fori_loop(0, CB, _edge, 0)
        pltpu.sync_copy(mb, oacc.at[didx[b]], add=True)

    _fire_idx(0, 0)
    _wait_idx(0)
    _fire_gathers(0, 0)
    _fire_idx(1, 1)

    def _pair(i, _):
        for b in range(2):
            ci = 2 * i + b

            @pl.when(ci < NCB)
            def _(ci=ci, b=b):
                nb = 1 - b

                @pl.when(ci + 1 < NCB)
                def _():
                    _wait_idx(nb)
                    _fire_gathers(ci + 1, nb)
                _wait_gathers(b)
                _compute(b)

                @pl.when(ci + 2 < NCB)
                def _():
                    _fire_idx(ci + 2, b)
        return 0
    lax.fori_loop(0, (NCB + 1) // 2, _pair, 0)

    plsc.subcore_barrier()

    pltpu.sync_copy(gbh.at[:, pl.ds(c * FH, FH)], gbbuf)
    for k in range(H):
        bv = jnp.zeros((L,), _f32)
        for h in range(H):
            bv = bv + gbbuf[h, pl.ds(k * L, L)]
        biasbuf[pl.ds(k * L, L)] = bv * (1.0 / H)

    for r in range(TPR // RW):
        off = s * TPR + r * RW

        @pl.when(off < N)
        def _(off=off):
            pltpu.sync_copy(oacc.at[pl.ds(off, RW)], msgbuf)

            def _brow(j, _):
                for k in range(H):
                    msgbuf[j, pl.ds(k * L, L)] = (msgbuf[j, pl.ds(k * L, L)]
                                                  + biasbuf[pl.ds(k * L, L)])
                return 0
            lax.fori_loop(0, RW, _brow, 0)
            pltpu.sync_copy(msgbuf, outh.at[pl.ds(c * N + off, RW)])

  return phase_a, phase_b


def kernel(x, edge_index, W1, b1, W2, attn_l, attn_r, gat_bias):
    phase_a, phase_b = _sc_kernels()
    fa, fb, elp, erp, mx = _dense(x, W1, b1, W2, attn_l, attn_r)
    src = edge_index[0]
    dst = edge_index[1]
    ee, pden = phase_a(elp, erp, mx, src, dst)
    rden = _rden(pden[:N], pden[N:])
    o2 = phase_b(fa, fb, ee, rden, src, dst, gat_bias)
    return jnp.concatenate([o2[:N], o2[N:]], axis=1)


# speedup vs baseline: 12.8865x; 1.1966x over previous
import functools

import jax
import jax.numpy as jnp
from jax import lax
from jax.experimental import pallas as pl
from jax.experimental.pallas import tpu as pltpu
from jax.experimental.pallas import tpu_sc as plsc

N = 10000
E = 160000
D_IN = 256
D_HID = 512
H = 8
F = 256
FH = 128
NEG = 0.2

NC = 2
NS = 16
L = 16

BN = 400
GRID_N = N // BN

EPW = E // (NC * NS)
CA = 40
NCA = EPW // CA

EPS = E // NS
CB = 16
NCB = EPS // CB

TPR = 640
ZCA = 40
RW = 16

_f32 = jnp.float32
_i32 = jnp.int32


def _dense_body(x_ref, w1_ref, b1_ref, w2_ref, al_ref, ar_ref,
                fa_ref, fb_ref, elp_ref, erp_ref, mx_ref):
    i = pl.program_id(0)
    h = jnp.maximum(
        jnp.dot(x_ref[...], w1_ref[...], preferred_element_type=_f32)
        + b1_ref[...], 0.0)
    feat = jnp.dot(h, w2_ref[...], preferred_element_type=_f32)
    ft = feat.reshape(BN, H, F)
    el = jnp.sum(ft * al_ref[...][None], axis=-1)
    er = jnp.sum(ft * ar_ref[...][None], axis=-1)
    z = jnp.zeros((BN, FH - H), _f32)
    elp_ref[...] = jnp.concatenate([el, z], axis=1)
    erp_ref[...] = jnp.concatenate([er, z], axis=1)
    fa_ref[...] = ft[:, :, :FH].reshape(BN, H * FH)
    fb_ref[...] = ft[:, :, FH:].reshape(BN, H * FH)
    bm = jnp.stack([jnp.full((FH,), jnp.max(el)), jnp.full((FH,), jnp.max(er))])
    prev = jnp.where(i == 0, jnp.full((2, FH), -jnp.inf, _f32), mx_ref[...])
    mx_ref[...] = jnp.maximum(prev, bm)


def _dense(x, W1, b1, W2, attn_l, attn_r):
    return pl.pallas_call(
        _dense_body,
        grid=(GRID_N,),
        in_specs=[
            pl.BlockSpec((BN, D_IN), lambda i: (i, 0)),
            pl.BlockSpec((D_IN, D_HID), lambda i: (0, 0)),
            pl.BlockSpec((1, D_HID), lambda i: (0, 0)),
            pl.BlockSpec((D_HID, H * F), lambda i: (0, 0)),
            pl.BlockSpec((H, F), lambda i: (0, 0)),
            pl.BlockSpec((H, F), lambda i: (0, 0)),
        ],
        out_specs=[
            pl.BlockSpec((BN, H * FH), lambda i: (i, 0)),
            pl.BlockSpec((BN, H * FH), lambda i: (i, 0)),
            pl.BlockSpec((BN, FH), lambda i: (i, 0)),
            pl.BlockSpec((BN, FH), lambda i: (i, 0)),
            pl.BlockSpec((2, FH), lambda i: (0, 0)),
        ],
        out_shape=[
            jax.ShapeDtypeStruct((N, H * FH), _f32),
            jax.ShapeDtypeStruct((N, H * FH), _f32),
            jax.ShapeDtypeStruct((N, FH), _f32),
            jax.ShapeDtypeStruct((N, FH), _f32),
            jax.ShapeDtypeStruct((2, FH), _f32),
        ],
    )(x, W1, b1.reshape(1, D_HID), W2, attn_l, attn_r)


def _rden_body(p0_ref, p1_ref, o_ref):
    o_ref[...] = (1.0 / H) / (p0_ref[...] + p1_ref[...] + 1e-30)


def _rden(p0, p1):
    return pl.pallas_call(
        _rden_body,
        grid=(5,),
        in_specs=[pl.BlockSpec((N // 5, FH), lambda i: (i, 0)),
                  pl.BlockSpec((N // 5, FH), lambda i: (i, 0))],
        out_specs=pl.BlockSpec((N // 5, FH), lambda i: (i, 0)),
        out_shape=jax.ShapeDtypeStruct((N, FH), _f32),
    )(p0, p1)


@functools.lru_cache(maxsize=None)
def _sc_kernels():
  mesh = plsc.VectorSubcoreMesh(core_axis_name="c", subcore_axis_name="s",
                                num_cores=NC, num_subcores=NS)

  @functools.partial(
      pl.kernel,
      out_type=[
          jax.ShapeDtypeStruct((E, L), _f32),
          jax.ShapeDtypeStruct((NC * N, FH), _f32),
      ],
      mesh=mesh,
      scratch_types=[
          pltpu.VMEM((CA,), _i32),
          pltpu.VMEM((CA,), _i32),
          pltpu.VMEM((CA,), _i32),
          pltpu.VMEM((CA,), _i32),
          pltpu.VMEM((CA, FH), _f32),
          pltpu.VMEM((CA, FH), _f32),
          pltpu.VMEM((CA, FH), _f32),
          pltpu.VMEM((CA, FH), _f32),
          pltpu.VMEM((CA, L), _f32),
          pltpu.VMEM((CA, L), _f32),
          pltpu.VMEM((CA, FH), _f32),
          pltpu.VMEM((2, FH), _f32),
          pltpu.VMEM((L,), _f32),
          pltpu.VMEM_SHARED((N, FH), _f32),
          pltpu.SemaphoreType.DMA,
          pltpu.SemaphoreType.DMA,
          pltpu.SemaphoreType.DMA,
          pltpu.SemaphoreType.DMA,
          pltpu.SemaphoreType.DMA,
          pltpu.SemaphoreType.DMA,
      ],
  )
  def phase_a(elp, erp, mx, srch, dsth, eeh, pdenh,
              sidx0, sidx1, didx0, didx1, elr0, elr1, err0, err1,
              eec0, eec1, eewide, mxbuf, bref, pdacc,
              isem0, isem1, gsem0, gsem1, wsem0, wsem1):
    c = lax.axis_index("c")
    s = lax.axis_index("s")
    w = s * NC + c
    sidx = [sidx0, sidx1]
    didx = [didx0, didx1]
    elr = [elr0, elr1]
    err_ = [err0, err1]
    eec = [eec0, eec1]
    isem = [isem0, isem1]
    gsem = [gsem0, gsem1]
    wsem = [wsem0, wsem1]

    pltpu.sync_copy(mx, mxbuf)
    bref[...] = mxbuf[0, pl.ds(0, L)] + mxbuf[1, pl.ds(0, L)]

    def _zrow(j, _):
        for k in range(FH // L):
            eewide[j, pl.ds(k * L, L)] = jnp.zeros((L,), _f32)
        return 0
    lax.fori_loop(0, CA, _zrow, 0)

    for r in range(TPR // ZCA):
        off = s * TPR + r * ZCA

        @pl.when(off < N)
        def _(off=off):
            pltpu.sync_copy(eewide.at[pl.ds(0, ZCA)],
                            pdacc.at[pl.ds(off, ZCA)])
    plsc.subcore_barrier()

    def _base(ci):
        return w * EPW + ci * CA

    def _fire_idx(ci, b):
        pltpu.async_copy(srch.at[pl.ds(_base(ci), CA)], sidx[b], isem[b])
        pltpu.async_copy(dsth.at[pl.ds(_base(ci), CA)], didx[b], isem[b])

    def _wait_idx(b):
        pltpu.make_async_copy(srch.at[pl.ds(0, CA)], sidx[b], isem[b]).wait()
        pltpu.make_async_copy(dsth.at[pl.ds(0, CA)], didx[b], isem[b]).wait()

    def _fire_gathers(b):
        pltpu.async_copy(elp.at[sidx[b]], elr[b], gsem[b])
        pltpu.async_copy(erp.at[didx[b]], err_[b], gsem[b])

    def _wait_gathers(b):
        pltpu.make_async_copy(elp.at[sidx[b]], elr[b], gsem[b]).wait()
        pltpu.make_async_copy(erp.at[didx[b]], err_[b], gsem[b]).wait()

    def _compute(ci, b):
        bv = bref[...]
        eebuf = eec[b]
        elb = elr[b]
        erb = err_[b]

        @pl.when(ci >= 2)
        def _():
            pltpu.make_async_copy(eebuf, eeh.at[pl.ds(0, CA)], wsem[b]).wait()

        def _edge(j, _):
            ev = elb[j, pl.ds(0, L)] + erb[j, pl.ds(0, L)]
            ee = jnp.exp(jnp.maximum(ev, NEG * ev) - bv)
            eebuf[j, :] = ee
            eewide[j, pl.ds(0, L)] = ee
            return 0
        lax.fori_loop(0, CA, _edge, 0)
        pltpu.async_copy(eebuf, eeh.at[pl.ds(_base(ci), CA)], wsem[b])
        pltpu.sync_copy(eewide, pdacc.at[didx[b]], add=True)

    _fire_idx(0, 0)
    _wait_idx(0)
    _fire_gathers(0)
    _fire_idx(1, 1)

    def _pair(i, _):
        for b in range(2):
            ci = 2 * i + b

            @pl.when(ci < NCA)
            def _(ci=ci, b=b):
                nb = 1 - b

                @pl.when(ci + 1 < NCA)
                def _():
                    _wait_idx(nb)
                    _fire_gathers(nb)
                _wait_gathers(b)
                _compute(ci, b)

                @pl.when(ci + 2 < NCA)
                def _():
                    _fire_idx(ci + 2, b)
        return 0
    lax.fori_loop(0, (NCA + 1) // 2, _pair, 0)

    pltpu.make_async_copy(eec[0], eeh.at[pl.ds(0, CA)], wsem[0]).wait()
    pltpu.make_async_copy(eec[1], eeh.at[pl.ds(0, CA)], wsem[1]).wait()

    plsc.subcore_barrier()
    for r in range(TPR // ZCA):
        off = s * TPR + r * ZCA

        @pl.when(off < N)
        def _(off=off):
            pltpu.sync_copy(pdacc.at[pl.ds(off, ZCA)],
                            pdenh.at[pl.ds(c * N + off, ZCA)])

  @functools.partial(
      pl.kernel,
      out_type=jax.ShapeDtypeStruct((NC * N, FH), _f32),
      mesh=mesh,
      scratch_types=[
          pltpu.VMEM((CB,), _i32),
          pltpu.VMEM((CB,), _i32),
          pltpu.VMEM((CB,), _i32),
          pltpu.VMEM((CB,), _i32),
          pltpu.VMEM((CB, L), _f32),
          pltpu.VMEM((CB, L), _f32),
          pltpu.VMEM((CB, FH), _f32),
          pltpu.VMEM((CB, FH), _f32),
          pltpu.VMEM((CB, H * FH), _f32),
          pltpu.VMEM((CB, H * FH), _f32),
          pltpu.VMEM((CB, FH), _f32),
          pltpu.VMEM((CB, FH), _f32),
          pltpu.VMEM((CB,), _i32),
          pltpu.VMEM((CB,), _i32),
          pltpu.VMEM((H, FH), _f32),
          pltpu.VMEM((FH,), _f32),
          pltpu.VMEM_SHARED((N, FH), _f32),
          pltpu.SemaphoreType.DMA,
          pltpu.SemaphoreType.DMA,
          pltpu.SemaphoreType.DMA,
          pltpu.SemaphoreType.DMA,
          pltpu.SemaphoreType.DMA,
          pltpu.SemaphoreType.DMA,
      ],
  )
  def phase_b(fah, fbh, eeh, rdenh, srch, dsth, gbh, outh,
              sidx0, sidx1, didx0, didx1, eeb0, eeb1, den0, den1,
              fb0, fb1, mb0, mb1, dsc0, dsc1, gbbuf, biasbuf, oacc,
              isem0, isem1, gsem0, gsem1, ssem0, ssem1):
    c = lax.axis_index("c")
    s = lax.axis_index("s")
    sidx = [sidx0, sidx1]
    didx = [didx0, didx1]
    eeb = [eeb0, eeb1]
    den = [den0, den1]
    fbuf = [fb0, fb1]
    mbuf = [mb0, mb1]
    isem = [isem0, isem1]
    gsem = [gsem0, gsem1]
    dsc = [dsc0, dsc1]
    ssem = [ssem0, ssem1]
    msgbuf = mb0

    def _zrow(j, _):
        for k in range(H):
            msgbuf[j, pl.ds(k * L, L)] = jnp.zeros((L,), _f32)
        return 0
    lax.fori_loop(0, RW, _zrow, 0)
    for r in range(TPR // RW):
        off = s * TPR + r * RW

        @pl.when(off < N)
        def _(off=off):
            pltpu.sync_copy(msgbuf, oacc.at[pl.ds(off, RW)])
    plsc.subcore_barrier()

    def _base(ci):
        return s * EPS + ci * CB

    def _fire_idx(ci, b):
        pltpu.async_copy(srch.at[pl.ds(_base(ci), CB)], sidx[b], isem[b])
        pltpu.async_copy(dsth.at[pl.ds(_base(ci), CB)], didx[b], isem[b])

    def _wait_idx(b):
        pltpu.make_async_copy(srch.at[pl.ds(0, CB)], sidx[b], isem[b]).wait()
        pltpu.make_async_copy(dsth.at[pl.ds(0, CB)], didx[b], isem[b]).wait()

    def _fire_gathers(ci, b):
        pltpu.async_copy(eeh.at[pl.ds(_base(ci), CB)], eeb[b], gsem[b])
        pltpu.async_copy(rdenh.at[didx[b]], den[b], gsem[b])

        @pl.when(c == 0)
        def _():
            pltpu.async_copy(fah.at[sidx[b]], fbuf[b], gsem[b])

        @pl.when(c == 1)
        def _():
            pltpu.async_copy(fbh.at[sidx[b]], fbuf[b], gsem[b])

    def _wait_gathers(b):
        pltpu.make_async_copy(eeh.at[pl.ds(0, CB)], eeb[b], gsem[b]).wait()
        pltpu.make_async_copy(rdenh.at[didx[b]], den[b], gsem[b]).wait()
        pltpu.make_async_copy(fah.at[sidx[b]], fbuf[b], gsem[b]).wait()

    def _compute(ci, b):
        featbuf = fbuf[b]
        mb = mbuf[b]
        eebuf = eeb[b]
        denb = den[b]

        @pl.when(ci >= 2)
        def _():
            pltpu.make_async_copy(mbuf[b], oacc.at[dsc[b]], ssem[b]).wait()

        def _edge(j, _):
            wv = eebuf[j] * denb[j, pl.ds(0, L)]
            for k in range(H):
                m = jnp.zeros((L,), _f32)
                for h in range(H):
                    sv = jnp.full((L,), wv[h], _f32)
                    m = m + sv * featbuf[j, pl.ds(h * FH + k * L, L)]
                mb[j, pl.ds(k * L, L)] = m
            return 0
        lax.fori_loop(0, CB, _edge, 0)
        dsc[b][...] = didx[b][...]
        pltpu.async_copy(mb, oacc.at[dsc[b]], ssem[b], add=True)

    _fire_idx(0, 0)
    _wait_idx(0)
    _fire_gathers(0, 0)
    _fire_idx(1, 1)

    def _pair(i, _):
        for b in range(2):
            ci = 2 * i + b

            @pl.when(ci < NCB)
            def _(ci=ci, b=b):
                nb = 1 - b

                @pl.when(ci + 1 < NCB)
                def _():
                    _wait_idx(nb)
                    _fire_gathers(ci + 1, nb)
                _wait_gathers(b)
                _compute(ci, b)

                @pl.when(ci + 2 < NCB)
                def _():
                    _fire_idx(ci + 2, b)
        return 0
    lax.fori_loop(0, (NCB + 1) // 2, _pair, 0)

    pltpu.make_async_copy(mbuf[0], oacc.at[dsc[0]], ssem[0]).wait()
    pltpu.make_async_copy(mbuf[1], oacc.at[dsc[1]], ssem[1]).wait()

    plsc.subcore_barrier()

    pltpu.sync_copy(gbh.at[:, pl.ds(c * FH, FH)], gbbuf)
    for k in range(H):
        bv = jnp.zeros((L,), _f32)
        for h in range(H):
            bv = bv + gbbuf[h, pl.ds(k * L, L)]
        biasbuf[pl.ds(k * L, L)] = bv * (1.0 / H)

    for r in range(TPR // RW):
        off = s * TPR + r * RW

        @pl.when(off < N)
        def _(off=off):
            pltpu.sync_copy(oacc.at[pl.ds(off, RW)], msgbuf)

            def _brow(j, _):
                for k in range(H):
                    msgbuf[j, pl.ds(k * L, L)] = (msgbuf[j, pl.ds(k * L, L)]
                                                  + biasbuf[pl.ds(k * L, L)])
                return 0
            lax.fori_loop(0, RW, _brow, 0)
            pltpu.sync_copy(msgbuf, outh.at[pl.ds(c * N + off, RW)])

  return phase_a, phase_b


def kernel(x, edge_index, W1, b1, W2, attn_l, attn_r, gat_bias):
    phase_a, phase_b = _sc_kernels()
    fa, fb, elp, erp, mx = _dense(x, W1, b1, W2, attn_l, attn_r)
    src = edge_index[0]
    dst = edge_index[1]
    ee, pden = phase_a(elp, erp, mx, src, dst)
    rden = _rden(pden[:N], pden[N:])
    o2 = phase_b(fa, fb, ee, rden, src, dst, gat_bias)
    return jnp.concatenate([o2[:N], o2[N:]], axis=1)


# speedup vs baseline: 13.6979x; 1.0630x over previous
import functools

import jax
import jax.numpy as jnp
from jax import lax
from jax.experimental import pallas as pl
from jax.experimental.pallas import tpu as pltpu
from jax.experimental.pallas import tpu_sc as plsc

N = 10000
E = 160000
D_IN = 256
D_HID = 512
H = 8
F = 256
FH = 128
NEG = 0.2

NC = 2
NS = 16
L = 16

BN = 400
GRID_N = N // BN

EPW = E // (NC * NS)
CA = 40
NCA = EPW // CA

CB = 32
NCHB = E // CB
NCBT = -(-NCHB // NS)

TPR = 640
ZCA = 40
RW = 16
WR = 80

_f32 = jnp.float32
_i32 = jnp.int32


def _dense_body(x_ref, w1_ref, b1_ref, w2_ref, al_ref, ar_ref,
                fa_ref, fb_ref, elp_ref, erp_ref, mx_ref):
    i = pl.program_id(0)
    h = jnp.maximum(
        jnp.dot(x_ref[...], w1_ref[...], preferred_element_type=_f32)
        + b1_ref[...], 0.0)
    feat = jnp.dot(h, w2_ref[...], preferred_element_type=_f32)
    ft = feat.reshape(BN, H, F)
    el = jnp.sum(ft * al_ref[...][None], axis=-1)
    er = jnp.sum(ft * ar_ref[...][None], axis=-1)
    z = jnp.zeros((BN, FH - H), _f32)
    elp_ref[...] = jnp.concatenate([el, z], axis=1)
    erp_ref[...] = jnp.concatenate([er, z], axis=1)
    q = jax.lax.bitcast_convert_type(
        ft.astype(jnp.bfloat16).astype(_f32), _i32)
    qa = q[:, :, :FH].reshape(BN, H * FH)
    qb = q[:, :, FH:].reshape(BN, H * FH)
    hw = H * FH // 2
    top = jnp.int32(-65536)
    fa_ref[...] = (jax.lax.shift_right_logical(qa[:, :hw], 16)
                   | (qa[:, hw:] & top))
    fb_ref[...] = (jax.lax.shift_right_logical(qb[:, :hw], 16)
                   | (qb[:, hw:] & top))
    bm = jnp.stack([jnp.full((FH,), jnp.max(el)), jnp.full((FH,), jnp.max(er))])
    prev = jnp.where(i == 0, jnp.full((2, FH), -jnp.inf, _f32), mx_ref[...])
    mx_ref[...] = jnp.maximum(prev, bm)


def _dense(x, W1, b1, W2, attn_l, attn_r):
    return pl.pallas_call(
        _dense_body,
        grid=(GRID_N,),
        in_specs=[
            pl.BlockSpec((BN, D_IN), lambda i: (i, 0)),
            pl.BlockSpec((D_IN, D_HID), lambda i: (0, 0)),
            pl.BlockSpec((1, D_HID), lambda i: (0, 0)),
            pl.BlockSpec((D_HID, H * F), lambda i: (0, 0)),
            pl.BlockSpec((H, F), lambda i: (0, 0)),
            pl.BlockSpec((H, F), lambda i: (0, 0)),
        ],
        out_specs=[
            pl.BlockSpec((BN, H * FH // 2), lambda i: (i, 0)),
            pl.BlockSpec((BN, H * FH // 2), lambda i: (i, 0)),
            pl.BlockSpec((BN, FH), lambda i: (i, 0)),
            pl.BlockSpec((BN, FH), lambda i: (i, 0)),
            pl.BlockSpec((2, FH), lambda i: (0, 0)),
        ],
        out_shape=[
            jax.ShapeDtypeStruct((N, H * FH // 2), _i32),
            jax.ShapeDtypeStruct((N, H * FH // 2), _i32),
            jax.ShapeDtypeStruct((N, FH), _f32),
            jax.ShapeDtypeStruct((N, FH), _f32),
            jax.ShapeDtypeStruct((2, FH), _f32),
        ],
    )(x, W1, b1.reshape(1, D_HID), W2, attn_l, attn_r)


def _rden_body(p0_ref, p1_ref, o_ref):
    o_ref[...] = (1.0 / H) / (p0_ref[...] + p1_ref[...] + 1e-30)


def _rden(p0, p1):
    return pl.pallas_call(
        _rden_body,
        grid=(5,),
        in_specs=[pl.BlockSpec((N // 5, FH), lambda i: (i, 0)),
                  pl.BlockSpec((N // 5, FH), lambda i: (i, 0))],
        out_specs=pl.BlockSpec((N // 5, FH), lambda i: (i, 0)),
        out_shape=jax.ShapeDtypeStruct((N, FH), _f32),
    )(p0, p1)


def _finish_body(o0_ref, o1_ref, gb_ref, out_ref):
    out_ref[...] = (jnp.concatenate([o0_ref[...], o1_ref[...]], axis=1)
                    + jnp.mean(gb_ref[...], axis=0))


def _finish(o0, o1, gb):
    return pl.pallas_call(
        _finish_body,
        grid=(5,),
        in_specs=[pl.BlockSpec((N // 5, FH), lambda i: (i, 0)),
                  pl.BlockSpec((N // 5, FH), lambda i: (i, 0)),
                  pl.BlockSpec((H, F), lambda i: (0, 0))],
        out_specs=pl.BlockSpec((N // 5, F), lambda i: (i, 0)),
        out_shape=jax.ShapeDtypeStruct((N, F), _f32),
    )(o0, o1, gb)


@functools.lru_cache(maxsize=None)
def _sc_kernels():
  mesh = plsc.VectorSubcoreMesh(core_axis_name="c", subcore_axis_name="s",
                                num_cores=NC, num_subcores=NS)

  @functools.partial(
      pl.kernel,
      out_type=[
          jax.ShapeDtypeStruct((E * L,), _f32),
          jax.ShapeDtypeStruct((NC * N, FH), _f32),
      ],
      mesh=mesh,
      scratch_types=[
          pltpu.VMEM((CA,), _i32),
          pltpu.VMEM((CA,), _i32),
          pltpu.VMEM((CA,), _i32),
          pltpu.VMEM((CA,), _i32),
          pltpu.VMEM((CA, FH), _f32),
          pltpu.VMEM((CA, FH), _f32),
          pltpu.VMEM((CA, FH), _f32),
          pltpu.VMEM((CA, FH), _f32),
          pltpu.VMEM((CA * L,), _f32),
          pltpu.VMEM((CA * L,), _f32),
          pltpu.VMEM((CA, FH), _f32),
          pltpu.VMEM((2, FH), _f32),
          pltpu.VMEM((L,), _f32),
          pltpu.VMEM_SHARED((N, FH), _f32),
          pltpu.SemaphoreType.DMA,
          pltpu.SemaphoreType.DMA,
          pltpu.SemaphoreType.DMA,
          pltpu.SemaphoreType.DMA,
          pltpu.SemaphoreType.DMA,
          pltpu.SemaphoreType.DMA,
      ],
  )
  def phase_a(elp, erp, mx, srch, dsth, eeh, pdenh,
              sidx0, sidx1, didx0, didx1, elr0, elr1, err0, err1,
              eec0, eec1, eewide, mxbuf, bref, pdacc,
              isem0, isem1, gsem0, gsem1, wsem0, wsem1):
    c = lax.axis_index("c")
    s = lax.axis_index("s")
    w = s * NC + c
    sidx = [sidx0, sidx1]
    didx = [didx0, didx1]
    elr = [elr0, elr1]
    err_ = [err0, err1]
    eec = [eec0, eec1]
    isem = [isem0, isem1]
    gsem = [gsem0, gsem1]
    wsem = [wsem0, wsem1]

    pltpu.sync_copy(mx, mxbuf)
    bref[...] = mxbuf[0, pl.ds(0, L)] + mxbuf[1, pl.ds(0, L)]

    def _zrow(j, _):
        for k in range(FH // L):
            eewide[j, pl.ds(k * L, L)] = jnp.zeros((L,), _f32)
        return 0
    lax.fori_loop(0, CA, _zrow, 0)

    for r in range(TPR // ZCA):
        off = s * TPR + r * ZCA

        @pl.when(off < N)
        def _(off=off):
            pltpu.sync_copy(eewide.at[pl.ds(0, ZCA)],
                            pdacc.at[pl.ds(off, ZCA)])
    plsc.subcore_barrier()

    def _base(ci):
        return w * EPW + ci * CA

    def _fire_idx(ci, b):
        pltpu.async_copy(srch.at[pl.ds(_base(ci), CA)], sidx[b], isem[b])
        pltpu.async_copy(dsth.at[pl.ds(_base(ci), CA)], didx[b], isem[b])

    def _wait_idx(b):
        pltpu.make_async_copy(srch.at[pl.ds(0, CA)], sidx[b], isem[b]).wait()
        pltpu.make_async_copy(dsth.at[pl.ds(0, CA)], didx[b], isem[b]).wait()

    def _fire_gathers(b):
        pltpu.async_copy(elp.at[sidx[b]], elr[b], gsem[b])
        pltpu.async_copy(erp.at[didx[b]], err_[b], gsem[b])

    def _wait_gathers(b):
        pltpu.make_async_copy(elp.at[sidx[b]], elr[b], gsem[b]).wait()
        pltpu.make_async_copy(erp.at[didx[b]], err_[b], gsem[b]).wait()

    def _compute(ci, b):
        bv = bref[...]
        eebuf = eec[b]
        elb = elr[b]
        erb = err_[b]

        @pl.when(ci >= 2)
        def _():
            pltpu.make_async_copy(eebuf, eeh.at[pl.ds(0, CA * L)],
                                  wsem[b]).wait()

        def _edge(j, _):
            ev = elb[j, pl.ds(0, L)] + erb[j, pl.ds(0, L)]
            ee = jnp.exp(jnp.maximum(ev, NEG * ev) - bv)
            eebuf[pl.ds(j * L, L)] = ee
            eewide[j, pl.ds(0, L)] = ee
            return 0
        lax.fori_loop(0, CA, _edge, 0)
        pltpu.async_copy(eebuf, eeh.at[pl.ds(_base(ci) * L, CA * L)],
                         wsem[b])
        pltpu.sync_copy(eewide, pdacc.at[didx[b]], add=True)

    _fire_idx(0, 0)
    _wait_idx(0)
    _fire_gathers(0)
    _fire_idx(1, 1)

    def _pair(i, _):
        for b in range(2):
            ci = 2 * i + b

            @pl.when(ci < NCA)
            def _(ci=ci, b=b):
                nb = 1 - b

                @pl.when(ci + 1 < NCA)
                def _():
                    _wait_idx(nb)
                    _fire_gathers(nb)
                _wait_gathers(b)
                _compute(ci, b)

                @pl.when(ci + 2 < NCA)
                def _():
                    _fire_idx(ci + 2, b)
        return 0
    lax.fori_loop(0, (NCA + 1) // 2, _pair, 0)

    pltpu.make_async_copy(eec[0], eeh.at[pl.ds(0, CA * L)], wsem[0]).wait()
    pltpu.make_async_copy(eec[1], eeh.at[pl.ds(0, CA * L)], wsem[1]).wait()

    plsc.subcore_barrier()
    for r in range(TPR // ZCA):
        off = s * TPR + r * ZCA

        @pl.when(off < N)
        def _(off=off):
            pltpu.sync_copy(pdacc.at[pl.ds(off, ZCA)],
                            pdenh.at[pl.ds(c * N + off, ZCA)])

  @functools.partial(
      pl.kernel,
      out_type=jax.ShapeDtypeStruct((NC * N, FH), _f32),
      mesh=mesh,
      scratch_types=[
          pltpu.VMEM((CB,), _i32),
          pltpu.VMEM((CB,), _i32),
          pltpu.VMEM((CB,), _i32),
          pltpu.VMEM((CB,), _i32),
          pltpu.VMEM((CB * L,), _f32),
          pltpu.VMEM((CB * L,), _f32),
          pltpu.VMEM((CB, FH), _f32),
          pltpu.VMEM((CB, FH), _f32),
          pltpu.VMEM((CB, H * FH // 2), _i32),
          pltpu.VMEM((CB, H * FH // 2), _i32),
          pltpu.VMEM((CB, FH), _f32),
          pltpu.VMEM((CB, FH), _f32),
          pltpu.VMEM((CB,), _i32),
          pltpu.VMEM((CB,), _i32),
          pltpu.VMEM_SHARED((N, FH), _f32),
          pltpu.SemaphoreType.DMA,
          pltpu.SemaphoreType.DMA,
          pltpu.SemaphoreType.DMA,
          pltpu.SemaphoreType.DMA,
          pltpu.SemaphoreType.DMA,
          pltpu.SemaphoreType.DMA,
      ],
  )
  def phase_b(fah, fbh, eeh, rdenh, srch, dsth, outh,
              sidx0, sidx1, didx0, didx1, eeb0, eeb1, den0, den1,
              fb0, fb1, mb0, mb1, dsc0, dsc1, oacc,
              isem0, isem1, gsem0, gsem1, ssem0, ssem1):
    c = lax.axis_index("c")
    s = lax.axis_index("s")
    sidx = [sidx0, sidx1]
    didx = [didx0, didx1]
    eeb = [eeb0, eeb1]
    den = [den0, den1]
    fbuf = [fb0, fb1]
    mbuf = [mb0, mb1]
    isem = [isem0, isem1]
    gsem = [gsem0, gsem1]
    dsc = [dsc0, dsc1]
    ssem = [ssem0, ssem1]
    msgbuf = mb0

    def _zrow(j, _):
        for k in range(H):
            msgbuf[j, pl.ds(k * L, L)] = jnp.zeros((L,), _f32)
        return 0
    lax.fori_loop(0, RW, _zrow, 0)
    for r in range(TPR // RW):
        off = s * TPR + r * RW

        @pl.when(off < N)
        def _(off=off):
            pltpu.sync_copy(msgbuf.at[pl.ds(0, RW)], oacc.at[pl.ds(off, RW)])
    plsc.subcore_barrier()

    def _base(ci):
        return (ci * NS + s) * CB

    def _fire_idx(ci, b):
        pltpu.async_copy(srch.at[pl.ds(_base(ci), CB)], sidx[b], isem[b])
        pltpu.async_copy(dsth.at[pl.ds(_base(ci), CB)], didx[b], isem[b])

    def _wait_idx(b):
        pltpu.make_async_copy(srch.at[pl.ds(0, CB)], sidx[b], isem[b]).wait()
        pltpu.make_async_copy(dsth.at[pl.ds(0, CB)], didx[b], isem[b]).wait()

    def _fire_gathers(ci, b):
        pltpu.async_copy(eeh.at[pl.ds(_base(ci) * L, CB * L)], eeb[b], gsem[b])
        pltpu.async_copy(rdenh.at[didx[b]], den[b], gsem[b])

        @pl.when(c == 0)
        def _():
            pltpu.async_copy(fah.at[sidx[b]], fbuf[b], gsem[b])

        @pl.when(c == 1)
        def _():
            pltpu.async_copy(fbh.at[sidx[b]], fbuf[b], gsem[b])

    def _wait_gathers(b):
        pltpu.make_async_copy(eeh.at[pl.ds(0, CB * L)], eeb[b], gsem[b]).wait()
        pltpu.make_async_copy(rdenh.at[didx[b]], den[b], gsem[b]).wait()
        pltpu.make_async_copy(fah.at[sidx[b]], fbuf[b], gsem[b]).wait()

    _TOP = jnp.int32(-65536)

    def _compute(ci, b):
        featbuf = fbuf[b]
        mb = mbuf[b]
        eebuf = eeb[b]
        denb = den[b]

        @pl.when(ci >= 2)
        def _():
            pltpu.make_async_copy(mbuf[b], oacc.at[dsc[b]], ssem[b]).wait()

        def _edge(j, _):
            wv = eebuf[pl.ds(j * L, L)] * denb[j, pl.ds(0, L)]
            for k in range(FH // L):
                m = jnp.zeros((L,), _f32)
                for h in range(H // 2):
                    sv_lo = jnp.full((L,), wv[h], _f32)
                    sv_hi = jnp.full((L,), wv[h + 4], _f32)
                    wrd = featbuf[j, pl.ds(h * FH + k * L, L)]
                    lo = jax.lax.bitcast_convert_type(wrd << 16, _f32)
                    hi = jax.lax.bitcast_convert_type(wrd & _TOP, _f32)
                    m = m + sv_lo * lo + sv_hi * hi
                mb[j, pl.ds(k * L, L)] = m
            return 0
        lax.fori_loop(0, CB, _edge, 0)
        dsc[b][...] = didx[b][...]
        pltpu.async_copy(mb, oacc.at[dsc[b]], ssem[b], add=True)

    _fire_idx(0, 0)
    _wait_idx(0)
    _fire_gathers(0, 0)
    _fire_idx(1, 1)

    def _pair(i, _):
        for b in range(2):
            ci = 2 * i + b

            @pl.when(ci * NS + s < NCHB)
            def _(ci=ci, b=b):
                nb = 1 - b

                @pl.when((ci + 1) * NS + s < NCHB)
                def _():
                    _wait_idx(nb)
                    _fire_gathers(ci + 1, nb)
                _wait_gathers(b)
                _compute(ci, b)

                @pl.when((ci + 2) * NS + s < NCHB)
                def _():
                    _fire_idx(ci + 2, b)
        return 0
    lax.fori_loop(0, (NCBT + 1) // 2, _pair, 0)

    pltpu.make_async_copy(mbuf[0], oacc.at[dsc[0]], ssem[0]).wait()
    pltpu.make_async_copy(mbuf[1], oacc.at[dsc[1]], ssem[1]).wait()

    plsc.subcore_barrier()

    for r in range(TPR // WR):
        off = s * TPR + r * WR

        @pl.when(off < N)
        def _(off=off):
            pltpu.sync_copy(oacc.at[pl.ds(off, WR)],
                            outh.at[pl.ds(c * N + off, WR)])

  return phase_a, phase_b


def kernel(x, edge_index, W1, b1, W2, attn_l, attn_r, gat_bias):
    phase_a, phase_b = _sc_kernels()
    fa, fb, elp, erp, mx = _dense(x, W1, b1, W2, attn_l, attn_r)
    src = edge_index[0]
    dst = edge_index[1]
    ee, pden = phase_a(elp, erp, mx, src, dst)
    rden = _rden(pden[:N], pden[N:])
    o2 = phase_b(fa, fb, ee, rden, src, dst)
    return _finish(o2[:N], o2[N:], gat_bias)


# speedup vs baseline: 13.7015x; 1.0003x over previous
import functools

import jax
import jax.numpy as jnp
from jax import lax
from jax.experimental import pallas as pl
from jax.experimental.pallas import tpu as pltpu
from jax.experimental.pallas import tpu_sc as plsc

N = 10000
E = 160000
D_IN = 256
D_HID = 512
H = 8
F = 256
FH = 128
NEG = 0.2

NC = 2
NS = 16
L = 16

BN = 400
GRID_N = N // BN

EPW = E // (NC * NS)
CA = 40
NCA = EPW // CA

CB = 32
NCHB = E // CB
NCBT = -(-NCHB // NS)

TPR = 640
ZCA = 40
RW = 16
WR = 80

_f32 = jnp.float32
_i32 = jnp.int32


def _dense_body(x_ref, w1_ref, b1_ref, w2_ref, al_ref, ar_ref,
                fa_ref, fb_ref, elp_ref, erp_ref, mx_ref):
    i = pl.program_id(0)
    h = jnp.maximum(
        jnp.dot(x_ref[...], w1_ref[...], preferred_element_type=_f32)
        + b1_ref[...], 0.0)
    feat = jnp.dot(h, w2_ref[...], preferred_element_type=_f32)
    ft = feat.reshape(BN, H, F)
    el = jnp.sum(ft * al_ref[...][None], axis=-1)
    er = jnp.sum(ft * ar_ref[...][None], axis=-1)
    z = jnp.zeros((BN, FH - H), _f32)
    elp_ref[...] = jnp.concatenate([el, z], axis=1)
    erp_ref[...] = jnp.concatenate([er, z], axis=1)
    q = jax.lax.bitcast_convert_type(
        ft.astype(jnp.bfloat16).astype(_f32), _i32)
    qa = q[:, :, :FH].reshape(BN, H * FH)
    qb = q[:, :, FH:].reshape(BN, H * FH)
    hw = H * FH // 2
    top = jnp.int32(-65536)
    fa_ref[...] = (jax.lax.shift_right_logical(qa[:, :hw], 16)
                   | (qa[:, hw:] & top))
    fb_ref[...] = (jax.lax.shift_right_logical(qb[:, :hw], 16)
                   | (qb[:, hw:] & top))
    bm = jnp.stack([jnp.full((FH,), jnp.max(el)), jnp.full((FH,), jnp.max(er))])
    prev = jnp.where(i == 0, jnp.full((2, FH), -jnp.inf, _f32), mx_ref[...])
    mx_ref[...] = jnp.maximum(prev, bm)


def _dense(x, W1, b1, W2, attn_l, attn_r):
    return pl.pallas_call(
        _dense_body,
        grid=(GRID_N,),
        in_specs=[
            pl.BlockSpec((BN, D_IN), lambda i: (i, 0)),
            pl.BlockSpec((D_IN, D_HID), lambda i: (0, 0)),
            pl.BlockSpec((1, D_HID), lambda i: (0, 0)),
            pl.BlockSpec((D_HID, H * F), lambda i: (0, 0)),
            pl.BlockSpec((H, F), lambda i: (0, 0)),
            pl.BlockSpec((H, F), lambda i: (0, 0)),
        ],
        out_specs=[
            pl.BlockSpec((BN, H * FH // 2), lambda i: (i, 0)),
            pl.BlockSpec((BN, H * FH // 2), lambda i: (i, 0)),
            pl.BlockSpec((BN, FH), lambda i: (i, 0)),
            pl.BlockSpec((BN, FH), lambda i: (i, 0)),
            pl.BlockSpec((2, FH), lambda i: (0, 0)),
        ],
        out_shape=[
            jax.ShapeDtypeStruct((N, H * FH // 2), _i32),
            jax.ShapeDtypeStruct((N, H * FH // 2), _i32),
            jax.ShapeDtypeStruct((N, FH), _f32),
            jax.ShapeDtypeStruct((N, FH), _f32),
            jax.ShapeDtypeStruct((2, FH), _f32),
        ],
    )(x, W1, b1.reshape(1, D_HID), W2, attn_l, attn_r)


def _rden_body(p0_ref, p1_ref, o_ref):
    o_ref[...] = (1.0 / H) / (p0_ref[...] + p1_ref[...] + 1e-30)


def _rden(p0, p1):
    return pl.pallas_call(
        _rden_body,
        grid=(5,),
        in_specs=[pl.BlockSpec((N // 5, FH), lambda i: (i, 0)),
                  pl.BlockSpec((N // 5, FH), lambda i: (i, 0))],
        out_specs=pl.BlockSpec((N // 5, FH), lambda i: (i, 0)),
        out_shape=jax.ShapeDtypeStruct((N, FH), _f32),
    )(p0, p1)


def _finish_body(o0_ref, o1_ref, gb_ref, out_ref):
    out_ref[...] = (jnp.concatenate([o0_ref[...], o1_ref[...]], axis=1)
                    + jnp.mean(gb_ref[...], axis=0))


def _finish(o0, o1, gb):
    return pl.pallas_call(
        _finish_body,
        grid=(5,),
        in_specs=[pl.BlockSpec((N // 5, FH), lambda i: (i, 0)),
                  pl.BlockSpec((N // 5, FH), lambda i: (i, 0)),
                  pl.BlockSpec((H, F), lambda i: (0, 0))],
        out_specs=pl.BlockSpec((N // 5, F), lambda i: (i, 0)),
        out_shape=jax.ShapeDtypeStruct((N, F), _f32),
    )(o0, o1, gb)


@functools.lru_cache(maxsize=None)
def _sc_kernels():
  mesh = plsc.VectorSubcoreMesh(core_axis_name="c", subcore_axis_name="s",
                                num_cores=NC, num_subcores=NS)

  @functools.partial(
      pl.kernel,
      out_type=[
          jax.ShapeDtypeStruct((E * L,), _f32),
          jax.ShapeDtypeStruct((NC * N, FH), _f32),
      ],
      mesh=mesh,
      scratch_types=[
          pltpu.VMEM((CA,), _i32),
          pltpu.VMEM((CA,), _i32),
          pltpu.VMEM((CA,), _i32),
          pltpu.VMEM((CA,), _i32),
          pltpu.VMEM((CA, FH), _f32),
          pltpu.VMEM((CA, FH), _f32),
          pltpu.VMEM((CA, FH), _f32),
          pltpu.VMEM((CA, FH), _f32),
          pltpu.VMEM((CA * L,), _f32),
          pltpu.VMEM((CA * L,), _f32),
          pltpu.VMEM((CA, FH), _f32),
          pltpu.VMEM((2, FH), _f32),
          pltpu.VMEM((L,), _f32),
          pltpu.VMEM_SHARED((N, FH), _f32),
          pltpu.SemaphoreType.DMA,
          pltpu.SemaphoreType.DMA,
          pltpu.SemaphoreType.DMA,
          pltpu.SemaphoreType.DMA,
          pltpu.SemaphoreType.DMA,
          pltpu.SemaphoreType.DMA,
      ],
  )
  def phase_a(elp, erp, mx, srch, dsth, eeh, pdenh,
              sidx0, sidx1, didx0, didx1, elr0, elr1, err0, err1,
              eec0, eec1, eewide, mxbuf, bref, pdacc,
              isem0, isem1, gsem0, gsem1, wsem0, wsem1):
    c = lax.axis_index("c")
    s = lax.axis_index("s")
    w = s * NC + c
    sidx = [sidx0, sidx1]
    didx = [didx0, didx1]
    elr = [elr0, elr1]
    err_ = [err0, err1]
    eec = [eec0, eec1]
    isem = [isem0, isem1]
    gsem = [gsem0, gsem1]
    wsem = [wsem0, wsem1]

    pltpu.sync_copy(mx, mxbuf)
    bref[...] = mxbuf[0, pl.ds(0, L)] + mxbuf[1, pl.ds(0, L)]

    def _zrow(j, _):
        for k in range(FH // L):
            eewide[j, pl.ds(k * L, L)] = jnp.zeros((L,), _f32)
        return 0
    lax.fori_loop(0, CA, _zrow, 0)

    for r in range(TPR // ZCA):
        off = s * TPR + r * ZCA

        @pl.when(off < N)
        def _(off=off):
            pltpu.sync_copy(eewide.at[pl.ds(0, ZCA)],
                            pdacc.at[pl.ds(off, ZCA)])
    plsc.subcore_barrier()

    def _base(ci):
        return w * EPW + ci * CA

    def _fire_idx(ci, b):
        pltpu.async_copy(srch.at[pl.ds(_base(ci), CA)], sidx[b], isem[b])
        pltpu.async_copy(dsth.at[pl.ds(_base(ci), CA)], didx[b], isem[b])

    def _wait_idx(b):
        pltpu.make_async_copy(srch.at[pl.ds(0, CA)], sidx[b], isem[b]).wait()
        pltpu.make_async_copy(dsth.at[pl.ds(0, CA)], didx[b], isem[b]).wait()

    def _fire_gathers(b):
        pltpu.async_copy(elp.at[sidx[b]], elr[b], gsem[b])
        pltpu.async_copy(erp.at[didx[b]], err_[b], gsem[b])

    def _wait_gathers(b):
        pltpu.make_async_copy(elp.at[sidx[b]], elr[b], gsem[b]).wait()
        pltpu.make_async_copy(erp.at[didx[b]], err_[b], gsem[b]).wait()

    def _compute(ci, b):
        bv = bref[...]
        eebuf = eec[b]
        elb = elr[b]
        erb = err_[b]

        @pl.when(ci >= 2)
        def _():
            pltpu.make_async_copy(eebuf, eeh.at[pl.ds(0, CA * L)],
                                  wsem[b]).wait()

        def _edge(j, _):
            ev = elb[j, pl.ds(0, L)] + erb[j, pl.ds(0, L)]
            ee = jnp.exp(jnp.maximum(ev, NEG * ev) - bv)
            eebuf[pl.ds(j * L, L)] = ee
            eewide[j, pl.ds(0, L)] = ee
            return 0
        lax.fori_loop(0, CA, _edge, 0)
        pltpu.async_copy(eebuf, eeh.at[pl.ds(_base(ci) * L, CA * L)],
                         wsem[b])
        pltpu.sync_copy(eewide, pdacc.at[didx[b]], add=True)

    _fire_idx(0, 0)
    _wait_idx(0)
    _fire_gathers(0)
    _fire_idx(1, 1)

    def _pair(i, _):
        for b in range(2):
            ci = 2 * i + b

            @pl.when(ci < NCA)
            def _(ci=ci, b=b):
                nb = 1 - b

                @pl.when(ci + 1 < NCA)
                def _():
                    _wait_idx(nb)
                    _fire_gathers(nb)
                _wait_gathers(b)
                _compute(ci, b)

                @pl.when(ci + 2 < NCA)
                def _():
                    _fire_idx(ci + 2, b)
        return 0
    lax.fori_loop(0, (NCA + 1) // 2, _pair, 0)

    pltpu.make_async_copy(eec[0], eeh.at[pl.ds(0, CA * L)], wsem[0]).wait()
    pltpu.make_async_copy(eec[1], eeh.at[pl.ds(0, CA * L)], wsem[1]).wait()

    plsc.subcore_barrier()
    for r in range(TPR // ZCA):
        off = s * TPR + r * ZCA

        @pl.when(off < N)
        def _(off=off):
            pltpu.sync_copy(pdacc.at[pl.ds(off, ZCA)],
                            pdenh.at[pl.ds(c * N + off, ZCA)])

  @functools.partial(
      pl.kernel,
      out_type=jax.ShapeDtypeStruct((NC * N, FH), _f32),
      mesh=mesh,
      scratch_types=[
          pltpu.VMEM((CB,), _i32),
          pltpu.VMEM((CB,), _i32),
          pltpu.VMEM((CB,), _i32),
          pltpu.VMEM((CB,), _i32),
          pltpu.VMEM((CB * L,), _f32),
          pltpu.VMEM((CB * L,), _f32),
          pltpu.VMEM((CB, FH), _f32),
          pltpu.VMEM((CB, FH), _f32),
          pltpu.VMEM((CB, H * FH // 2), _i32),
          pltpu.VMEM((CB, H * FH // 2), _i32),
          pltpu.VMEM((CB, FH), _f32),
          pltpu.VMEM((CB, FH), _f32),
          pltpu.VMEM((CB,), _i32),
          pltpu.VMEM((CB,), _i32),
          pltpu.VMEM_SHARED((N, FH), _f32),
          pltpu.SemaphoreType.DMA,
          pltpu.SemaphoreType.DMA,
          pltpu.SemaphoreType.DMA,
          pltpu.SemaphoreType.DMA,
          pltpu.SemaphoreType.DMA,
          pltpu.SemaphoreType.DMA,
      ],
  )
  def phase_b(fah, fbh, eeh, rdenh, srch, dsth, outh,
              sidx0, sidx1, didx0, didx1, eeb0, eeb1, den0, den1,
              fb0, fb1, mb0, mb1, dsc0, dsc1, oacc,
              isem0, isem1, gsem0, gsem1, ssem0, ssem1):
    c = lax.axis_index("c")
    s = lax.axis_index("s")
    sidx = [sidx0, sidx1]
    didx = [didx0, didx1]
    eeb = [eeb0, eeb1]
    den = [den0, den1]
    fbuf = [fb0, fb1]
    mbuf = [mb0, mb1]
    isem = [isem0, isem1]
    gsem = [gsem0, gsem1]
    dsc = [dsc0, dsc1]
    ssem = [ssem0, ssem1]
    msgbuf = mb0

    def _zrow(j, _):
        for k in range(H):
            msgbuf[j, pl.ds(k * L, L)] = jnp.zeros((L,), _f32)
        return 0
    lax.fori_loop(0, RW, _zrow, 0)
    for r in range(TPR // RW):
        off = s * TPR + r * RW

        @pl.when(off < N)
        def _(off=off):
            pltpu.sync_copy(msgbuf.at[pl.ds(0, RW)], oacc.at[pl.ds(off, RW)])
    plsc.subcore_barrier()

    def _base(ci):
        return (ci * NS + s) * CB

    def _fire_idx(ci, b):
        pltpu.async_copy(srch.at[pl.ds(_base(ci), CB)], sidx[b], isem[b])
        pltpu.async_copy(dsth.at[pl.ds(_base(ci), CB)], didx[b], isem[b])

    def _wait_idx(b):
        pltpu.make_async_copy(srch.at[pl.ds(0, CB)], sidx[b], isem[b]).wait()
        pltpu.make_async_copy(dsth.at[pl.ds(0, CB)], didx[b], isem[b]).wait()

    def _fire_gathers(ci, b):
        pltpu.async_copy(eeh.at[pl.ds(_base(ci) * L, CB * L)], eeb[b], gsem[b])
        pltpu.async_copy(rdenh.at[didx[b]], den[b], gsem[b])

        @pl.when(c == 0)
        def _():
            pltpu.async_copy(fah.at[sidx[b]], fbuf[b], gsem[b])

        @pl.when(c == 1)
        def _():
            pltpu.async_copy(fbh.at[sidx[b]], fbuf[b], gsem[b])

    def _wait_gathers(b):
        pltpu.make_async_copy(eeh.at[pl.ds(0, CB * L)], eeb[b], gsem[b]).wait()
        pltpu.make_async_copy(rdenh.at[didx[b]], den[b], gsem[b]).wait()
        pltpu.make_async_copy(fah.at[sidx[b]], fbuf[b], gsem[b]).wait()

    _TOP = jnp.int32(-65536)
    _GDN = jax.lax.GatherDimensionNumbers(
        offset_dims=(), collapsed_slice_dims=(0,), start_index_map=(0,))

    def _blane(v, h):
        idx = jnp.full((L, 1), h, _i32)
        return jax.lax.gather(
            v, idx, _GDN, (1,),
            mode=jax.lax.GatherScatterMode.PROMISE_IN_BOUNDS)

    def _compute(ci, b):
        featbuf = fbuf[b]
        mb = mbuf[b]
        eebuf = eeb[b]
        denb = den[b]

        @pl.when(ci >= 2)
        def _():
            pltpu.make_async_copy(mbuf[b], oacc.at[dsc[b]], ssem[b]).wait()

        def _edge(j, _):
            wv = eebuf[pl.ds(j * L, L)] * denb[j, pl.ds(0, L)]
            sv = [_blane(wv, h) for h in range(H)]
            for k in range(FH // L):
                m = jnp.zeros((L,), _f32)
                for h in range(H // 2):
                    wrd = featbuf[j, pl.ds(h * FH + k * L, L)]
                    lo = jax.lax.bitcast_convert_type(wrd << 16, _f32)
                    hi = jax.lax.bitcast_convert_type(wrd & _TOP, _f32)
                    m = m + sv[h] * lo + sv[h + 4] * hi
                mb[j, pl.ds(k * L, L)] = m
            return 0
        lax.fori_loop(0, CB, _edge, 0)
        dsc[b][...] = didx[b][...]
        pltpu.async_copy(mb, oacc.at[dsc[b]], ssem[b], add=True)

    _fire_idx(0, 0)
    _wait_idx(0)
    _fire_gathers(0, 0)
    _fire_idx(1, 1)

    def _pair(i, _):
        for b in range(2):
            ci = 2 * i + b

            @pl.when(ci * NS + s < NCHB)
            def _(ci=ci, b=b):
                nb = 1 - b

                @pl.when((ci + 1) * NS + s < NCHB)
                def _():
                    _wait_idx(nb)
                    _fire_gathers(ci + 1, nb)
                _wait_gathers(b)
                _compute(ci, b)

                @pl.when((ci + 2) * NS + s < NCHB)
                def _():
                    _fire_idx(ci + 2, b)
        return 0
    lax.fori_loop(0, (NCBT + 1) // 2, _pair, 0)

    pltpu.make_async_copy(mbuf[0], oacc.at[dsc[0]], ssem[0]).wait()
    pltpu.make_async_copy(mbuf[1], oacc.at[dsc[1]], ssem[1]).wait()

    plsc.subcore_barrier()

    for r in range(TPR // WR):
        off = s * TPR + r * WR

        @pl.when(off < N)
        def _(off=off):
            pltpu.sync_copy(oacc.at[pl.ds(off, WR)],
                            outh.at[pl.ds(c * N + off, WR)])

  return phase_a, phase_b


def kernel(x, edge_index, W1, b1, W2, attn_l, attn_r, gat_bias):
    phase_a, phase_b = _sc_kernels()
    fa, fb, elp, erp, mx = _dense(x, W1, b1, W2, attn_l, attn_r)
    src = edge_index[0]
    dst = edge_index[1]
    ee, pden = phase_a(elp, erp, mx, src, dst)
    rden = _rden(pden[:N], pden[N:])
    o2 = phase_b(fa, fb, ee, rden, src, dst)
    return _finish(o2[:N], o2[N:], gat_bias)


# speedup vs baseline: 13.7428x; 1.0030x over previous
import functools

import jax
import jax.numpy as jnp
from jax import lax
from jax.experimental import pallas as pl
from jax.experimental.pallas import tpu as pltpu
from jax.experimental.pallas import tpu_sc as plsc

N = 10000
E = 160000
D_IN = 256
D_HID = 512
H = 8
F = 256
FH = 128
NEG = 0.2

NC = 2
NS = 16
L = 16

BN = 400
GRID_N = N // BN

EPW = E // (NC * NS)
CA = 40
NCA = EPW // CA

CB = 32
NCHB = E // CB
NCBT = -(-NCHB // NS)

TPR = 640
ZCA = 40
RW = 16
WR = 80

_f32 = jnp.float32
_i32 = jnp.int32


def _dense_body(x_ref, w1_ref, b1_ref, w2_ref, al_ref, ar_ref,
                fa_ref, fb_ref, elp_ref, erp_ref, mx_ref):
    i = pl.program_id(0)
    h = jnp.maximum(
        jnp.dot(x_ref[...], w1_ref[...], preferred_element_type=_f32)
        + b1_ref[...], 0.0)
    feat = jnp.dot(h, w2_ref[...], preferred_element_type=_f32)
    ft = feat.reshape(BN, H, F)
    el = jnp.sum(ft * al_ref[...][None], axis=-1)
    er = jnp.sum(ft * ar_ref[...][None], axis=-1)
    z = jnp.zeros((BN, FH - H), _f32)
    elp_ref[...] = jnp.concatenate([el, z], axis=1)
    erp_ref[...] = jnp.concatenate([er, z], axis=1)
    q = jax.lax.bitcast_convert_type(
        ft.astype(jnp.bfloat16).astype(_f32), _i32)
    qa = q[:, :, :FH].reshape(BN, H * FH)
    qb = q[:, :, FH:].reshape(BN, H * FH)
    hw = H * FH // 2
    top = jnp.int32(-65536)
    fa_ref[...] = (jax.lax.shift_right_logical(qa[:, :hw], 16)
                   | (qa[:, hw:] & top))
    fb_ref[...] = (jax.lax.shift_right_logical(qb[:, :hw], 16)
                   | (qb[:, hw:] & top))
    bm = jnp.stack([jnp.full((FH,), jnp.max(el)), jnp.full((FH,), jnp.max(er))])
    prev = jnp.where(i == 0, jnp.full((2, FH), -jnp.inf, _f32), mx_ref[...])
    mx_ref[...] = jnp.maximum(prev, bm)


def _dense(x, W1, b1, W2, attn_l, attn_r):
    return pl.pallas_call(
        _dense_body,
        grid=(GRID_N,),
        in_specs=[
            pl.BlockSpec((BN, D_IN), lambda i: (i, 0)),
            pl.BlockSpec((D_IN, D_HID), lambda i: (0, 0)),
            pl.BlockSpec((1, D_HID), lambda i: (0, 0)),
            pl.BlockSpec((D_HID, H * F), lambda i: (0, 0)),
            pl.BlockSpec((H, F), lambda i: (0, 0)),
            pl.BlockSpec((H, F), lambda i: (0, 0)),
        ],
        out_specs=[
            pl.BlockSpec((BN, H * FH // 2), lambda i: (i, 0)),
            pl.BlockSpec((BN, H * FH // 2), lambda i: (i, 0)),
            pl.BlockSpec((BN, FH), lambda i: (i, 0)),
            pl.BlockSpec((BN, FH), lambda i: (i, 0)),
            pl.BlockSpec((2, FH), lambda i: (0, 0)),
        ],
        out_shape=[
            jax.ShapeDtypeStruct((N, H * FH // 2), _i32),
            jax.ShapeDtypeStruct((N, H * FH // 2), _i32),
            jax.ShapeDtypeStruct((N, FH), _f32),
            jax.ShapeDtypeStruct((N, FH), _f32),
            jax.ShapeDtypeStruct((2, FH), _f32),
        ],
    )(x, W1, b1.reshape(1, D_HID), W2, attn_l, attn_r)


def _rden_body(p0_ref, p1_ref, o_ref):
    o_ref[...] = (1.0 / H) / (p0_ref[...] + p1_ref[...] + 1e-30)


def _rden(p0, p1):
    return pl.pallas_call(
        _rden_body,
        grid=(5,),
        in_specs=[pl.BlockSpec((N // 5, FH), lambda i: (i, 0)),
                  pl.BlockSpec((N // 5, FH), lambda i: (i, 0))],
        out_specs=pl.BlockSpec((N // 5, FH), lambda i: (i, 0)),
        out_shape=jax.ShapeDtypeStruct((N, FH), _f32),
    )(p0, p1)


def _finish_body(o0_ref, o1_ref, gb_ref, out_ref):
    out_ref[...] = (jnp.concatenate([o0_ref[...], o1_ref[...]], axis=1)
                    + jnp.mean(gb_ref[...], axis=0))


def _finish(o0, o1, gb):
    return pl.pallas_call(
        _finish_body,
        grid=(5,),
        in_specs=[pl.BlockSpec((N // 5, FH), lambda i: (i, 0)),
                  pl.BlockSpec((N // 5, FH), lambda i: (i, 0)),
                  pl.BlockSpec((H, F), lambda i: (0, 0))],
        out_specs=pl.BlockSpec((N // 5, F), lambda i: (i, 0)),
        out_shape=jax.ShapeDtypeStruct((N, F), _f32),
    )(o0, o1, gb)


@functools.lru_cache(maxsize=None)
def _sc_kernels():
  mesh = plsc.VectorSubcoreMesh(core_axis_name="c", subcore_axis_name="s",
                                num_cores=NC, num_subcores=NS)

  @functools.partial(
      pl.kernel,
      out_type=[
          jax.ShapeDtypeStruct((E * L,), _f32),
          jax.ShapeDtypeStruct((NC * N, FH), _f32),
      ],
      mesh=mesh,
      scratch_types=[
          pltpu.VMEM((CA,), _i32),
          pltpu.VMEM((CA,), _i32),
          pltpu.VMEM((CA,), _i32),
          pltpu.VMEM((CA,), _i32),
          pltpu.VMEM((CA, FH), _f32),
          pltpu.VMEM((CA, FH), _f32),
          pltpu.VMEM((CA, FH), _f32),
          pltpu.VMEM((CA, FH), _f32),
          pltpu.VMEM((CA * L,), _f32),
          pltpu.VMEM((CA * L,), _f32),
          pltpu.VMEM((CA, FH), _f32),
          pltpu.VMEM((2, FH), _f32),
          pltpu.VMEM((L,), _f32),
          pltpu.VMEM_SHARED((N, FH), _f32),
          pltpu.SemaphoreType.DMA,
          pltpu.SemaphoreType.DMA,
          pltpu.SemaphoreType.DMA,
          pltpu.SemaphoreType.DMA,
          pltpu.SemaphoreType.DMA,
          pltpu.SemaphoreType.DMA,
      ],
  )
  def phase_a(elp, erp, mx, srch, dsth, eeh, pdenh,
              sidx0, sidx1, didx0, didx1, elr0, elr1, err0, err1,
              eec0, eec1, eewide, mxbuf, bref, pdacc,
              isem0, isem1, gsem0, gsem1, wsem0, wsem1):
    c = lax.axis_index("c")
    s = lax.axis_index("s")
    w = s * NC + c
    sidx = [sidx0, sidx1]
    didx = [didx0, didx1]
    elr = [elr0, elr1]
    err_ = [err0, err1]
    eec = [eec0, eec1]
    isem = [isem0, isem1]
    gsem = [gsem0, gsem1]
    wsem = [wsem0, wsem1]

    pltpu.sync_copy(mx, mxbuf)
    bref[...] = mxbuf[0, pl.ds(0, L)] + mxbuf[1, pl.ds(0, L)]

    def _zrow(j, _):
        for k in range(FH // L):
            eewide[j, pl.ds(k * L, L)] = jnp.zeros((L,), _f32)
        return 0
    lax.fori_loop(0, CA, _zrow, 0)

    for r in range(TPR // ZCA):
        off = s * TPR + r * ZCA

        @pl.when(off < N)
        def _(off=off):
            pltpu.sync_copy(eewide.at[pl.ds(0, ZCA)],
                            pdacc.at[pl.ds(off, ZCA)])
    plsc.subcore_barrier()

    def _base(ci):
        return w * EPW + ci * CA

    def _fire_idx(ci, b):
        pltpu.async_copy(srch.at[pl.ds(_base(ci), CA)], sidx[b], isem[b])
        pltpu.async_copy(dsth.at[pl.ds(_base(ci), CA)], didx[b], isem[b])

    def _wait_idx(b):
        pltpu.make_async_copy(srch.at[pl.ds(0, CA)], sidx[b], isem[b]).wait()
        pltpu.make_async_copy(dsth.at[pl.ds(0, CA)], didx[b], isem[b]).wait()

    def _fire_gathers(b):
        pltpu.async_copy(elp.at[sidx[b]], elr[b], gsem[b])
        pltpu.async_copy(erp.at[didx[b]], err_[b], gsem[b])

    def _wait_gathers(b):
        pltpu.make_async_copy(elp.at[sidx[b]], elr[b], gsem[b]).wait()
        pltpu.make_async_copy(erp.at[didx[b]], err_[b], gsem[b]).wait()

    def _compute(ci, b):
        bv = bref[...]
        eebuf = eec[b]
        elb = elr[b]
        erb = err_[b]

        @pl.when(ci >= 2)
        def _():
            pltpu.make_async_copy(eebuf, eeh.at[pl.ds(0, CA * L)],
                                  wsem[b]).wait()

        def _edge(j, _):
            ev = elb[j, pl.ds(0, L)] + erb[j, pl.ds(0, L)]
            ee = jnp.exp(jnp.maximum(ev, NEG * ev) - bv)
            eebuf[pl.ds(j * L, L)] = ee
            eewide[j, pl.ds(0, L)] = ee
            return 0
        lax.fori_loop(0, CA, _edge, 0)
        pltpu.async_copy(eebuf, eeh.at[pl.ds(_base(ci) * L, CA * L)],
                         wsem[b])
        pltpu.sync_copy(eewide, pdacc.at[didx[b]], add=True)

    _fire_idx(0, 0)
    _wait_idx(0)
    _fire_gathers(0)
    _fire_idx(1, 1)

    def _pair(i, _):
        for b in range(2):
            ci = 2 * i + b

            @pl.when(ci < NCA)
            def _(ci=ci, b=b):
                nb = 1 - b

                @pl.when(ci + 1 < NCA)
                def _():
                    _wait_idx(nb)
                    _fire_gathers(nb)
                _wait_gathers(b)
                _compute(ci, b)

                @pl.when(ci + 2 < NCA)
                def _():
                    _fire_idx(ci + 2, b)
        return 0
    lax.fori_loop(0, (NCA + 1) // 2, _pair, 0)

    pltpu.make_async_copy(eec[0], eeh.at[pl.ds(0, CA * L)], wsem[0]).wait()
    pltpu.make_async_copy(eec[1], eeh.at[pl.ds(0, CA * L)], wsem[1]).wait()

    plsc.subcore_barrier()
    for r in range(TPR // ZCA):
        off = s * TPR + r * ZCA

        @pl.when(off < N)
        def _(off=off):
            pltpu.sync_copy(pdacc.at[pl.ds(off, ZCA)],
                            pdenh.at[pl.ds(c * N + off, ZCA)])

  @functools.partial(
      pl.kernel,
      out_type=jax.ShapeDtypeStruct((NC * N, FH), _f32),
      mesh=mesh,
      scratch_types=[
          pltpu.VMEM((CB,), _i32),
          pltpu.VMEM((CB,), _i32),
          pltpu.VMEM((CB,), _i32),
          pltpu.VMEM((CB,), _i32),
          pltpu.VMEM((CB * L,), _f32),
          pltpu.VMEM((CB * L,), _f32),
          pltpu.VMEM((CB, FH), _f32),
          pltpu.VMEM((CB, FH), _f32),
          pltpu.VMEM((CB, H * FH // 2), _i32),
          pltpu.VMEM((CB, H * FH // 2), _i32),
          pltpu.VMEM((CB, FH), _f32),
          pltpu.VMEM((CB, FH), _f32),
          pltpu.VMEM((CB,), _i32),
          pltpu.VMEM((CB,), _i32),
          pltpu.VMEM_SHARED((N, FH), _f32),
          pltpu.SemaphoreType.DMA,
          pltpu.SemaphoreType.DMA,
          pltpu.SemaphoreType.DMA,
          pltpu.SemaphoreType.DMA,
          pltpu.SemaphoreType.DMA,
          pltpu.SemaphoreType.DMA,
      ],
  )
  def phase_b(fah, fbh, eeh, rdenh, srch, dsth, outh,
              sidx0, sidx1, didx0, didx1, eeb0, eeb1, den0, den1,
              fb0, fb1, mb0, mb1, dsc0, dsc1, oacc,
              isem0, isem1, gsem0, gsem1, ssem0, ssem1):
    c = lax.axis_index("c")
    s = lax.axis_index("s")
    sidx = [sidx0, sidx1]
    didx = [didx0, didx1]
    eeb = [eeb0, eeb1]
    den = [den0, den1]
    fbuf = [fb0, fb1]
    mbuf = [mb0, mb1]
    isem = [isem0, isem1]
    gsem = [gsem0, gsem1]
    dsc = [dsc0, dsc1]
    ssem = [ssem0, ssem1]
    msgbuf = mb0

    def _zrow(j, _):
        for k in range(H):
            msgbuf[j, pl.ds(k * L, L)] = jnp.zeros((L,), _f32)
        return 0
    lax.fori_loop(0, RW, _zrow, 0)
    for r in range(TPR // RW):
        off = s * TPR + r * RW

        @pl.when(off < N)
        def _(off=off):
            pltpu.sync_copy(msgbuf.at[pl.ds(0, RW)], oacc.at[pl.ds(off, RW)])
    plsc.subcore_barrier()

    def _base(ci):
        return (ci * NS + s) * CB

    def _fire_idx(ci, b):
        pltpu.async_copy(srch.at[pl.ds(_base(ci), CB)], sidx[b], isem[b])
        pltpu.async_copy(dsth.at[pl.ds(_base(ci), CB)], didx[b], isem[b])

    def _wait_idx(b):
        pltpu.make_async_copy(srch.at[pl.ds(0, CB)], sidx[b], isem[b]).wait()
        pltpu.make_async_copy(dsth.at[pl.ds(0, CB)], didx[b], isem[b]).wait()

    def _fire_gathers(ci, b):
        pltpu.async_copy(eeh.at[pl.ds(_base(ci) * L, CB * L)], eeb[b], gsem[b])
        pltpu.async_copy(rdenh.at[didx[b]], den[b], gsem[b])

        @pl.when(c == 0)
        def _():
            pltpu.async_copy(fah.at[sidx[b]], fbuf[b], gsem[b])

        @pl.when(c == 1)
        def _():
            pltpu.async_copy(fbh.at[sidx[b]], fbuf[b], gsem[b])

    def _wait_gathers(b):
        pltpu.make_async_copy(eeh.at[pl.ds(0, CB * L)], eeb[b], gsem[b]).wait()
        pltpu.make_async_copy(rdenh.at[didx[b]], den[b], gsem[b]).wait()
        pltpu.make_async_copy(fah.at[sidx[b]], fbuf[b], gsem[b]).wait()

    _TOP = jnp.int32(-65536)
    _GDN = jax.lax.GatherDimensionNumbers(
        offset_dims=(), collapsed_slice_dims=(0,), start_index_map=(0,))

    def _blane(v, h):
        idx = jnp.full((L, 1), h, _i32)
        return jax.lax.gather(
            v, idx, _GDN, (1,),
            mode=jax.lax.GatherScatterMode.PROMISE_IN_BOUNDS)

    def _compute(ci, b):
        featbuf = fbuf[b]
        mb = mbuf[b]
        eebuf = eeb[b]
        denb = den[b]

        @pl.when(ci >= 2)
        def _():
            pltpu.make_async_copy(mbuf[b], oacc.at[dsc[b]], ssem[b]).wait()

        def _one(j):
            wv = eebuf[pl.ds(j * L, L)] * denb[j, pl.ds(0, L)]
            sv = [_blane(wv, h) for h in range(H)]
            for k in range(FH // L):
                m = jnp.zeros((L,), _f32)
                for h in range(H // 2):
                    wrd = featbuf[j, pl.ds(h * FH + k * L, L)]
                    lo = jax.lax.bitcast_convert_type(wrd << 16, _f32)
                    hi = jax.lax.bitcast_convert_type(wrd & _TOP, _f32)
                    m = m + sv[h] * lo + sv[h + 4] * hi
                mb[j, pl.ds(k * L, L)] = m

        def _edge(jj, _):
            _one(2 * jj)
            _one(2 * jj + 1)
            return 0
        lax.fori_loop(0, CB // 2, _edge, 0)
        dsc[b][...] = didx[b][...]
        pltpu.async_copy(mb, oacc.at[dsc[b]], ssem[b], add=True)

    _fire_idx(0, 0)
    _wait_idx(0)
    _fire_gathers(0, 0)
    _fire_idx(1, 1)

    def _pair(i, _):
        for b in range(2):
            ci = 2 * i + b

            @pl.when(ci * NS + s < NCHB)
            def _(ci=ci, b=b):
                nb = 1 - b

                @pl.when((ci + 1) * NS + s < NCHB)
                def _():
                    _wait_idx(nb)
                    _fire_gathers(ci + 1, nb)
                _wait_gathers(b)
                _compute(ci, b)

                @pl.when((ci + 2) * NS + s < NCHB)
                def _():
                    _fire_idx(ci + 2, b)
        return 0
    lax.fori_loop(0, (NCBT + 1) // 2, _pair, 0)

    pltpu.make_async_copy(mbuf[0], oacc.at[dsc[0]], ssem[0]).wait()
    pltpu.make_async_copy(mbuf[1], oacc.at[dsc[1]], ssem[1]).wait()

    plsc.subcore_barrier()

    for r in range(TPR // WR):
        off = s * TPR + r * WR

        @pl.when(off < N)
        def _(off=off):
            pltpu.sync_copy(oacc.at[pl.ds(off, WR)],
                            outh.at[pl.ds(c * N + off, WR)])

  return phase_a, phase_b


def kernel(x, edge_index, W1, b1, W2, attn_l, attn_r, gat_bias):
    phase_a, phase_b = _sc_kernels()
    fa, fb, elp, erp, mx = _dense(x, W1, b1, W2, attn_l, attn_r)
    src = edge_index[0]
    dst = edge_index[1]
    ee, pden = phase_a(elp, erp, mx, src, dst)
    rden = _rden(pden[:N], pden[N:])
    o2 = phase_b(fa, fb, ee, rden, src, dst)
    return _finish(o2[:N], o2[N:], gat_bias)
